# Initial kernel scaffold; baseline (speedup 1.0000x reference)
import jax
import jax.numpy as jnp
from jax import lax
from jax.experimental import pallas as pl
from jax.experimental.pallas import tpu as pltpu
from jax.experimental.pallas import tpu_sc as plsc

N = 10000
NPAD = 10240
E = 320000
NC, NS = 2, 16
NW = NC * NS
CHUNK = 128
CPT = 79
EPW = CPT * CHUNK
EPAD = EPW * NW
ROWS2D = EPAD // CHUNK
STRIPE = NPAD // NS
PAD_SRC = N
PAD_DST = N + 100

_mesh = plsc.VectorSubcoreMesh(core_axis_name="c", subcore_axis_name="s")


def _fill_vmem(ref, rows, width, value):
    kpr = width // 16

    def body(i, _):
        r = i // kpr
        k = i % kpr
        ref[r, pl.ds(k * 16, 16)] = jnp.full((16,), value, jnp.float32)
        return 0

    lax.fori_loop(0, rows * kpr, body, 0)


def _deg_body(dst_hbm, out_hbm, dstv, onesv, zrow, acc, sem):
    del sem
    c = lax.axis_index("c")
    s = lax.axis_index("s")
    w = c * NS + s
    _fill_vmem(onesv, 1, CHUNK, 1.0)
    _fill_vmem(zrow, STRIPE // 16, 16, 0.0)
    pltpu.sync_copy(zrow, acc.at[pl.ds(s * STRIPE, STRIPE)])
    plsc.subcore_barrier()
    pltpu.sync_copy(dst_hbm.at[pl.ds(w * CPT, CPT)], dstv)

    def body(j, _):
        pltpu.sync_copy(onesv.at[0], acc.at[dstv.at[j]], add=True)
        return 0

    lax.fori_loop(0, CPT, body, 0)
    plsc.subcore_barrier()
    pltpu.sync_copy(acc.at[pl.ds(s * STRIPE, STRIPE)],
                    out_hbm.at[c, pl.ds(s * STRIPE, STRIPE)])


def _degree_kernel(dst2d):
    return pl.kernel(
        _deg_body,
        out_type=jax.ShapeDtypeStruct((NC, NPAD), jnp.float32),
        mesh=_mesh,
        scratch_types=[
            pltpu.VMEM((CPT, CHUNK), jnp.int32),
            pltpu.VMEM((1, CHUNK), jnp.float32),
            pltpu.VMEM((STRIPE // 16, 16), jnp.float32),
            pltpu.VMEM_SHARED((NPAD,), jnp.float32),
            pltpu.SemaphoreType.DMA,
        ],
    )(dst2d)


def _prop_body(g_hbm, src_hbm, dst_hbm, out_hbm, srcv, dstv, rows, zbuf, acc,
               sem):
    c = lax.axis_index("c")
    s = lax.axis_index("s")
    w = c * NS + s
    d = zbuf.shape[1]
    _fill_vmem(zbuf, CHUNK, d, 0.0)

    def zs(i, _):
        pltpu.sync_copy(zbuf, acc.at[pl.ds(s * STRIPE + i * CHUNK, CHUNK)])
        return 0

    lax.fori_loop(0, STRIPE // CHUNK, zs, 0)
    plsc.subcore_barrier()
    pltpu.sync_copy(src_hbm.at[pl.ds(w * CPT, CPT)], srcv)
    pltpu.sync_copy(dst_hbm.at[pl.ds(w * CPT, CPT)], dstv)

    def body(j, _):
        pltpu.async_copy(g_hbm.at[srcv.at[j]], rows, sem).wait()
        pltpu.sync_copy(rows, acc.at[dstv.at[j]], add=True)
        return 0

    lax.fori_loop(0, CPT, body, 0)
    plsc.subcore_barrier()
    pltpu.sync_copy(acc.at[pl.ds(s * STRIPE, STRIPE)],
                    out_hbm.at[c, pl.ds(s * STRIPE, STRIPE)])


def _propagate(g, src2d, dst2d, d):
    return pl.kernel(
        _prop_body,
        out_type=jax.ShapeDtypeStruct((NC, NPAD, d), jnp.float32),
        mesh=_mesh,
        scratch_types=[
            pltpu.VMEM((CPT, CHUNK), jnp.int32),
            pltpu.VMEM((CPT, CHUNK), jnp.int32),
            pltpu.VMEM((CHUNK, d), jnp.float32),
            pltpu.VMEM((CHUNK, d), jnp.float32),
            pltpu.VMEM_SHARED((NPAD, d), jnp.float32),
            pltpu.SemaphoreType.DMA,
        ],
    )(g, src2d, dst2d)


_MBLK = 1024
_GRID = NPAD // _MBLK


def _col_spec():
    return pl.BlockSpec((_MBLK, 1), lambda i: (i, 0))


def _mat_spec(d):
    return pl.BlockSpec((_MBLK, d), lambda i: (i, 0))


def _full_spec(r, c):
    return pl.BlockSpec((r, c), lambda i: (0, 0))


def _tc1_body(p0, p1, m, x, w1, g1, dv):
    deg = p0[...] + p1[...] + m[...]
    dinv = jnp.where(deg > 0, lax.rsqrt(deg), 0.0)
    g1[...] = dinv * jnp.dot(x[...], w1[...],
                             preferred_element_type=jnp.float32)
    dv[...] = dinv


def _tc1(p0, p1, m, x, w1):
    return pl.pallas_call(
        _tc1_body,
        grid=(_GRID,),
        in_specs=[_col_spec(), _col_spec(), _col_spec(), _mat_spec(128),
                  _full_spec(128, 128)],
        out_specs=[_mat_spec(128), _col_spec()],
        out_shape=[jax.ShapeDtypeStruct((NPAD, 128), jnp.float32),
                   jax.ShapeDtypeStruct((NPAD, 1), jnp.float32)],
    )(p0, p1, m, x, w1)


def _tc2_body(q0, q1, g1, dv, b1, w2, g2):
    dinv = dv[...]
    h1 = jnp.maximum(dinv * (q0[...] + q1[...] + g1[...]) + b1[...], 0.0)
    g2[...] = dinv * jnp.dot(h1, w2[...], preferred_element_type=jnp.float32)


def _tc2(q0, q1, g1, dv, b1, w2):
    return pl.pallas_call(
        _tc2_body,
        grid=(_GRID,),
        in_specs=[_mat_spec(128), _mat_spec(128), _mat_spec(128), _col_spec(),
                  _full_spec(1, 128), _full_spec(128, 64)],
        out_specs=_mat_spec(64),
        out_shape=jax.ShapeDtypeStruct((NPAD, 64), jnp.float32),
    )(q0, q1, g1, dv, b1, w2)


def _tc3_body(r0, r1, g2, dv, b2, out):
    out[...] = jnp.maximum(
        dv[...] * (r0[...] + r1[...] + g2[...]) + b2[...], 0.0)


def _tc3(r0, r1, g2, dv, b2):
    return pl.pallas_call(
        _tc3_body,
        grid=(_GRID,),
        in_specs=[_mat_spec(64), _mat_spec(64), _mat_spec(64), _col_spec(),
                  _full_spec(1, 64)],
        out_specs=_mat_spec(64),
        out_shape=jax.ShapeDtypeStruct((NPAD, 64), jnp.float32),
    )(r0, r1, g2, dv, b2)


@jax.jit
def kernel(x, edge_index, W1, b1, W2, b2):
    src = edge_index[0].astype(jnp.int32)
    dst = edge_index[1].astype(jnp.int32)
    pad = EPAD - E
    src2d = jnp.concatenate(
        [src, jnp.full((pad,), PAD_SRC, jnp.int32)]).reshape(ROWS2D, CHUNK)
    dst2d = jnp.concatenate(
        [dst, jnp.full((pad,), PAD_DST, jnp.int32)]).reshape(ROWS2D, CHUNK)
    xp = jnp.pad(x, ((0, NPAD - N), (0, 0)))
    maskc = (jnp.arange(NPAD) < N).astype(jnp.float32).reshape(NPAD, 1)

    degp = _degree_kernel(dst2d)
    p0 = degp[0].reshape(NPAD, 1)
    p1 = degp[1].reshape(NPAD, 1)
    g1, dv = _tc1(p0, p1, maskc, xp, W1)

    s1 = _propagate(g1, src2d, dst2d, 128)
    g2 = _tc2(s1[0], s1[1], g1, dv, b1.reshape(1, 128), W2)

    s2 = _propagate(g2, src2d, dst2d, 64)
    out = _tc3(s2[0], s2[1], g2, dv, b2.reshape(1, 64))
    return out[:N]


# speedup vs baseline: 10.1825x; 10.1825x over previous
import jax
import jax.numpy as jnp
from jax import lax
from jax.experimental import pallas as pl
from jax.experimental.pallas import tpu as pltpu
from jax.experimental.pallas import tpu_sc as plsc

N = 10000
NPAD = 10240
E = 320000
NC, NS = 2, 16
NW = NC * NS
CHUNK = 128
CPT = 80
EPW = CPT * CHUNK
EPAD = EPW * NW
ROWS2D = EPAD // CHUNK
STRIPE = NPAD // NS
PAD_SRC = N
PAD_DST = N + 100

_mesh = plsc.VectorSubcoreMesh(core_axis_name="c", subcore_axis_name="s")


def _fill_vmem(ref, rows, width, value):
    kpr = width // 16

    def body(i, _):
        r = i // kpr
        k = i % kpr
        ref[r, pl.ds(k * 16, 16)] = jnp.full((16,), value, jnp.float32)
        return 0

    lax.fori_loop(0, rows * kpr, body, 0)


def _fill_vmem1d(ref, n, value):
    def body(i, _):
        ref[pl.ds(i * 16, 16)] = jnp.full((16,), value, jnp.float32)
        return 0

    lax.fori_loop(0, n // 16, body, 0)


def _deg_body(dst_hbm, out_hbm, dstv, onesv, zrow, acc, sem):
    del sem
    c = lax.axis_index("c")
    s = lax.axis_index("s")
    w = c * NS + s
    _fill_vmem(onesv, 1, CHUNK, 1.0)
    _fill_vmem1d(zrow, STRIPE, 0.0)
    pltpu.sync_copy(zrow, acc.at[pl.ds(s * STRIPE, STRIPE)])
    plsc.subcore_barrier()
    pltpu.sync_copy(dst_hbm.at[pl.ds(w * CPT, CPT)], dstv)

    def body(j, _):
        pltpu.sync_copy(onesv.at[0], acc.at[dstv.at[j]], add=True)
        return 0

    lax.fori_loop(0, CPT, body, 0)
    plsc.subcore_barrier()
    pltpu.sync_copy(acc.at[pl.ds(s * STRIPE, STRIPE)],
                    out_hbm.at[c, pl.ds(s * STRIPE, STRIPE)])


def _degree_kernel(dst2d):
    return pl.kernel(
        _deg_body,
        out_type=jax.ShapeDtypeStruct((NC, NPAD), jnp.float32),
        mesh=_mesh,
        scratch_types=[
            pltpu.VMEM((CPT, CHUNK), jnp.int32),
            pltpu.VMEM((1, CHUNK), jnp.float32),
            pltpu.VMEM((STRIPE,), jnp.float32),
            pltpu.VMEM_SHARED((NPAD,), jnp.float32),
            pltpu.SemaphoreType.DMA,
        ],
    )(dst2d)


IDXG = 8


def _prop_body(g_hbm, src_hbm, dst_hbm, out_hbm, srcv, dstv, rows, acc, sem):
    c = lax.axis_index("c")
    s = lax.axis_index("s")
    w = c * NS + s
    d = rows.shape[1]
    _fill_vmem(rows, CHUNK, d, 0.0)

    def zs(i, _):
        pltpu.sync_copy(rows, acc.at[pl.ds(s * STRIPE + i * CHUNK, CHUNK)])
        return 0

    lax.fori_loop(0, STRIPE // CHUNK, zs, 0)
    plsc.subcore_barrier()

    def group(jg, _):
        base = w * CPT + jg * IDXG
        pltpu.sync_copy(src_hbm.at[pl.ds(base, IDXG)], srcv)
        pltpu.sync_copy(dst_hbm.at[pl.ds(base, IDXG)], dstv)

        def body(jj, _):
            pltpu.async_copy(g_hbm.at[srcv.at[jj]], rows, sem).wait()
            pltpu.sync_copy(rows, acc.at[dstv.at[jj]], add=True)
            return 0

        lax.fori_loop(0, IDXG, body, 0)
        return 0

    lax.fori_loop(0, CPT // IDXG, group, 0)
    plsc.subcore_barrier()
    pltpu.sync_copy(acc.at[pl.ds(s * STRIPE, STRIPE)],
                    out_hbm.at[c, pl.ds(s * STRIPE, STRIPE)])


def _propagate(g, src2d, dst2d, d):
    return pl.kernel(
        _prop_body,
        out_type=jax.ShapeDtypeStruct((NC, NPAD, d), jnp.float32),
        mesh=_mesh,
        scratch_types=[
            pltpu.VMEM((IDXG, CHUNK), jnp.int32),
            pltpu.VMEM((IDXG, CHUNK), jnp.int32),
            pltpu.VMEM((CHUNK, d), jnp.float32),
            pltpu.VMEM_SHARED((NPAD, d), jnp.float32),
            pltpu.SemaphoreType.DMA,
        ],
        compiler_params=pltpu.CompilerParams(use_tc_tiling_on_sc=False),
    )(g, src2d, dst2d)


_MBLK = 1024
_GRID = NPAD // _MBLK


def _col_spec():
    return pl.BlockSpec((_MBLK, 1), lambda i: (i, 0))


def _mat_spec(d):
    return pl.BlockSpec((_MBLK, d), lambda i: (i, 0))


def _full_spec(r, c):
    return pl.BlockSpec((r, c), lambda i: (0, 0))


def _tc1_body(p0, p1, m, x, w1, g1, dv):
    deg = p0[...] + p1[...] + m[...]
    dinv = jnp.where(deg > 0, lax.rsqrt(deg), 0.0)
    g1[...] = dinv * jnp.dot(x[...], w1[...],
                             preferred_element_type=jnp.float32)
    dv[...] = dinv


def _tc1(p0, p1, m, x, w1):
    return pl.pallas_call(
        _tc1_body,
        grid=(_GRID,),
        in_specs=[_col_spec(), _col_spec(), _col_spec(), _mat_spec(128),
                  _full_spec(128, 128)],
        out_specs=[_mat_spec(128), _col_spec()],
        out_shape=[jax.ShapeDtypeStruct((NPAD, 128), jnp.float32),
                   jax.ShapeDtypeStruct((NPAD, 1), jnp.float32)],
    )(p0, p1, m, x, w1)


def _tc2_body(q0, q1, g1, dv, b1, w2, g2):
    dinv = dv[...]
    h1 = jnp.maximum(dinv * (q0[...] + q1[...] + g1[...]) + b1[...], 0.0)
    g2[...] = dinv * jnp.dot(h1, w2[...], preferred_element_type=jnp.float32)


def _tc2(q0, q1, g1, dv, b1, w2):
    return pl.pallas_call(
        _tc2_body,
        grid=(_GRID,),
        in_specs=[_mat_spec(128), _mat_spec(128), _mat_spec(128), _col_spec(),
                  _full_spec(1, 128), _full_spec(128, 64)],
        out_specs=_mat_spec(64),
        out_shape=jax.ShapeDtypeStruct((NPAD, 64), jnp.float32),
    )(q0, q1, g1, dv, b1, w2)


def _tc3_body(r0, r1, g2, dv, b2, out):
    out[...] = jnp.maximum(
        dv[...] * (r0[...] + r1[...] + g2[...]) + b2[...], 0.0)


def _tc3(r0, r1, g2, dv, b2):
    return pl.pallas_call(
        _tc3_body,
        grid=(_GRID,),
        in_specs=[_mat_spec(64), _mat_spec(64), _mat_spec(64), _col_spec(),
                  _full_spec(1, 64)],
        out_specs=_mat_spec(64),
        out_shape=jax.ShapeDtypeStruct((NPAD, 64), jnp.float32),
    )(r0, r1, g2, dv, b2)


@jax.jit
def kernel(x, edge_index, W1, b1, W2, b2):
    src = edge_index[0].astype(jnp.int32)
    dst = edge_index[1].astype(jnp.int32)
    pad = EPAD - E
    src2d = jnp.concatenate(
        [src, jnp.full((pad,), PAD_SRC, jnp.int32)]).reshape(ROWS2D, CHUNK)
    dst2d = jnp.concatenate(
        [dst, jnp.full((pad,), PAD_DST, jnp.int32)]).reshape(ROWS2D, CHUNK)
    xp = jnp.pad(x, ((0, NPAD - N), (0, 0)))
    maskc = (jnp.arange(NPAD) < N).astype(jnp.float32).reshape(NPAD, 1)

    degp = _degree_kernel(dst2d)
    p0 = degp[0].reshape(NPAD, 1)
    p1 = degp[1].reshape(NPAD, 1)
    g1, dv = _tc1(p0, p1, maskc, xp, W1)

    s1 = _propagate(g1, src2d, dst2d, 128)
    g2 = _tc2(s1[0], s1[1], g1, dv, b1.reshape(1, 128), W2)

    s2 = _propagate(g2, src2d, dst2d, 64)
    out = _tc3(s2[0], s2[1], g2, dv, b2.reshape(1, 64))
    return out[:N]


# speedup vs baseline: 11.1819x; 1.0981x over previous
import jax
import jax.numpy as jnp
from jax import lax
from jax.experimental import pallas as pl
from jax.experimental.pallas import tpu as pltpu
from jax.experimental.pallas import tpu_sc as plsc

N = 10000
NPAD = 10240
E = 320000
NC, NS = 2, 16
NW = NC * NS
CHUNK = 128
CPT = 80
EPW = CPT * CHUNK
EPAD = EPW * NW
ROWS2D = EPAD // CHUNK
STRIPE = NPAD // NS
PAD_SRC = N
PAD_DST = N + 100

_mesh = plsc.VectorSubcoreMesh(core_axis_name="c", subcore_axis_name="s")


def _fill_vmem(ref, rows, width, value):
    kpr = width // 16

    def body(i, _):
        r = i // kpr
        k = i % kpr
        ref[r, pl.ds(k * 16, 16)] = jnp.full((16,), value, jnp.float32)
        return 0

    lax.fori_loop(0, rows * kpr, body, 0)


def _fill_vmem1d(ref, n, value):
    def body(i, _):
        ref[pl.ds(i * 16, 16)] = jnp.full((16,), value, jnp.float32)
        return 0

    lax.fori_loop(0, n // 16, body, 0)


def _deg_body(dst_hbm, out_hbm, dstv, onesv, zrow, acc, sem):
    del sem
    c = lax.axis_index("c")
    s = lax.axis_index("s")
    w = c * NS + s
    _fill_vmem(onesv, 1, CHUNK, 1.0)
    _fill_vmem1d(zrow, STRIPE, 0.0)
    pltpu.sync_copy(zrow, acc.at[pl.ds(s * STRIPE, STRIPE)])
    plsc.subcore_barrier()
    pltpu.sync_copy(dst_hbm.at[pl.ds(w * CPT, CPT)], dstv)

    def body(j, _):
        pltpu.sync_copy(onesv.at[0], acc.at[dstv.at[j]], add=True)
        return 0

    lax.fori_loop(0, CPT, body, 0)
    plsc.subcore_barrier()
    pltpu.sync_copy(acc.at[pl.ds(s * STRIPE, STRIPE)],
                    out_hbm.at[c, pl.ds(s * STRIPE, STRIPE)])


def _degree_kernel(dst2d):
    return pl.kernel(
        _deg_body,
        out_type=jax.ShapeDtypeStruct((NC, NPAD), jnp.float32),
        mesh=_mesh,
        scratch_types=[
            pltpu.VMEM((CPT, CHUNK), jnp.int32),
            pltpu.VMEM((1, CHUNK), jnp.float32),
            pltpu.VMEM((STRIPE,), jnp.float32),
            pltpu.VMEM_SHARED((NPAD,), jnp.float32),
            pltpu.SemaphoreType.DMA,
        ],
    )(dst2d)


IDXG = 8


def _prop_body(g_hbm, src_hbm, dst_hbm, out_hbm, srcv, dstv, rows, rows2, acc,
               sem, sem2):
    c = lax.axis_index("c")
    s = lax.axis_index("s")
    w = c * NS + s
    d = rows.shape[1]
    _fill_vmem(rows, CHUNK, d, 0.0)

    def zs(i, _):
        pltpu.sync_copy(rows, acc.at[pl.ds(s * STRIPE + i * CHUNK, CHUNK)])
        return 0

    lax.fori_loop(0, STRIPE // CHUNK, zs, 0)
    plsc.subcore_barrier()

    def group(jg, _):
        base = w * CPT + jg * IDXG
        pltpu.sync_copy(src_hbm.at[pl.ds(base, IDXG)], srcv)
        pltpu.sync_copy(dst_hbm.at[pl.ds(base, IDXG)], dstv)
        bufs = (rows, rows2)
        sems = (sem, sem2)
        descs = [None, None]
        descs[0] = pltpu.async_copy(g_hbm.at[srcv.at[0]], bufs[0], sems[0])
        for jj in range(IDXG):
            b = jj % 2
            if jj + 1 < IDXG:
                nb = (jj + 1) % 2
                descs[nb] = pltpu.async_copy(g_hbm.at[srcv.at[jj + 1]],
                                             bufs[nb], sems[nb])
            descs[b].wait()
            pltpu.sync_copy(bufs[b], acc.at[dstv.at[jj]], add=True)
        return 0

    lax.fori_loop(0, CPT // IDXG, group, 0)
    plsc.subcore_barrier()
    pltpu.sync_copy(acc.at[pl.ds(s * STRIPE, STRIPE)],
                    out_hbm.at[c, pl.ds(s * STRIPE, STRIPE)])


def _propagate(g, src2d, dst2d, d):
    return pl.kernel(
        _prop_body,
        out_type=jax.ShapeDtypeStruct((NC, NPAD, d), jnp.float32),
        mesh=_mesh,
        scratch_types=[
            pltpu.VMEM((IDXG, CHUNK), jnp.int32),
            pltpu.VMEM((IDXG, CHUNK), jnp.int32),
            pltpu.VMEM((CHUNK, d), jnp.float32),
            pltpu.VMEM((CHUNK, d), jnp.float32),
            pltpu.VMEM_SHARED((NPAD, d), jnp.float32),
            pltpu.SemaphoreType.DMA,
            pltpu.SemaphoreType.DMA,
        ],
        compiler_params=pltpu.CompilerParams(use_tc_tiling_on_sc=False),
    )(g, src2d, dst2d)


_MBLK = 1024
_GRID = NPAD // _MBLK


def _col_spec():
    return pl.BlockSpec((_MBLK, 1), lambda i: (i, 0))


def _mat_spec(d):
    return pl.BlockSpec((_MBLK, d), lambda i: (i, 0))


def _full_spec(r, c):
    return pl.BlockSpec((r, c), lambda i: (0, 0))


def _tc1_body(p0, p1, m, x, w1, g1, dv):
    deg = p0[...] + p1[...] + m[...]
    dinv = jnp.where(deg > 0, lax.rsqrt(deg), 0.0)
    g1[...] = dinv * jnp.dot(x[...], w1[...],
                             preferred_element_type=jnp.float32)
    dv[...] = dinv


def _tc1(p0, p1, m, x, w1):
    return pl.pallas_call(
        _tc1_body,
        grid=(_GRID,),
        in_specs=[_col_spec(), _col_spec(), _col_spec(), _mat_spec(128),
                  _full_spec(128, 128)],
        out_specs=[_mat_spec(128), _col_spec()],
        out_shape=[jax.ShapeDtypeStruct((NPAD, 128), jnp.float32),
                   jax.ShapeDtypeStruct((NPAD, 1), jnp.float32)],
    )(p0, p1, m, x, w1)


def _tc2_body(q0, q1, g1, dv, b1, w2, g2):
    dinv = dv[...]
    h1 = jnp.maximum(dinv * (q0[...] + q1[...] + g1[...]) + b1[...], 0.0)
    g2[...] = dinv * jnp.dot(h1, w2[...], preferred_element_type=jnp.float32)


def _tc2(q0, q1, g1, dv, b1, w2):
    return pl.pallas_call(
        _tc2_body,
        grid=(_GRID,),
        in_specs=[_mat_spec(128), _mat_spec(128), _mat_spec(128), _col_spec(),
                  _full_spec(1, 128), _full_spec(128, 64)],
        out_specs=_mat_spec(64),
        out_shape=jax.ShapeDtypeStruct((NPAD, 64), jnp.float32),
    )(q0, q1, g1, dv, b1, w2)


def _tc3_body(r0, r1, g2, dv, b2, out):
    out[...] = jnp.maximum(
        dv[...] * (r0[...] + r1[...] + g2[...]) + b2[...], 0.0)


def _tc3(r0, r1, g2, dv, b2):
    return pl.pallas_call(
        _tc3_body,
        grid=(_GRID,),
        in_specs=[_mat_spec(64), _mat_spec(64), _mat_spec(64), _col_spec(),
                  _full_spec(1, 64)],
        out_specs=_mat_spec(64),
        out_shape=jax.ShapeDtypeStruct((NPAD, 64), jnp.float32),
    )(r0, r1, g2, dv, b2)


@jax.jit
def kernel(x, edge_index, W1, b1, W2, b2):
    src = edge_index[0].astype(jnp.int32)
    dst = edge_index[1].astype(jnp.int32)
    pad = EPAD - E
    src2d = jnp.concatenate(
        [src, jnp.full((pad,), PAD_SRC, jnp.int32)]).reshape(ROWS2D, CHUNK)
    dst2d = jnp.concatenate(
        [dst, jnp.full((pad,), PAD_DST, jnp.int32)]).reshape(ROWS2D, CHUNK)
    xp = jnp.pad(x, ((0, NPAD - N), (0, 0)))
    maskc = (jnp.arange(NPAD) < N).astype(jnp.float32).reshape(NPAD, 1)

    degp = _degree_kernel(dst2d)
    p0 = degp[0].reshape(NPAD, 1)
    p1 = degp[1].reshape(NPAD, 1)
    g1, dv = _tc1(p0, p1, maskc, xp, W1)

    s1 = _propagate(g1, src2d, dst2d, 128)
    g2 = _tc2(s1[0], s1[1], g1, dv, b1.reshape(1, 128), W2)

    s2 = _propagate(g2, src2d, dst2d, 64)
    out = _tc3(s2[0], s2[1], g2, dv, b2.reshape(1, 64))
    return out[:N]


# speedup vs baseline: 11.1914x; 1.0009x over previous
import jax
import jax.numpy as jnp
from jax import lax
from jax.experimental import pallas as pl
from jax.experimental.pallas import tpu as pltpu
from jax.experimental.pallas import tpu_sc as plsc

N = 10000
NPAD = 10240
E = 320000
NC, NS = 2, 16
NW = NC * NS
CHUNK = 128
CPT = 80
EPW = CPT * CHUNK
EPAD = EPW * NW
ROWS2D = EPAD // CHUNK
STRIPE = NPAD // NS
PAD_SRC = N
PAD_DST = N + 100

_mesh = plsc.VectorSubcoreMesh(core_axis_name="c", subcore_axis_name="s")


def _fill_vmem(ref, rows, width, value):
    kpr = width // 16

    def body(i, _):
        r = i // kpr
        k = i % kpr
        ref[r, pl.ds(k * 16, 16)] = jnp.full((16,), value, jnp.float32)
        return 0

    lax.fori_loop(0, rows * kpr, body, 0)


def _fill_vmem1d(ref, n, value):
    def body(i, _):
        ref[pl.ds(i * 16, 16)] = jnp.full((16,), value, jnp.float32)
        return 0

    lax.fori_loop(0, n // 16, body, 0)


def _deg_body(dst_hbm, out_hbm, dstv, onesv, zrow, acc, sem):
    del sem
    c = lax.axis_index("c")
    s = lax.axis_index("s")
    w = c * NS + s
    _fill_vmem(onesv, 1, CHUNK, 1.0)
    _fill_vmem1d(zrow, STRIPE, 0.0)
    pltpu.sync_copy(zrow, acc.at[pl.ds(s * STRIPE, STRIPE)])
    plsc.subcore_barrier()
    pltpu.sync_copy(dst_hbm.at[pl.ds(w * CPT, CPT)], dstv)

    def body(j, _):
        pltpu.sync_copy(onesv.at[0], acc.at[dstv.at[j]], add=True)
        return 0

    lax.fori_loop(0, CPT, body, 0)
    plsc.subcore_barrier()
    pltpu.sync_copy(acc.at[pl.ds(s * STRIPE, STRIPE)],
                    out_hbm.at[c, pl.ds(s * STRIPE, STRIPE)])


def _degree_kernel(dst2d):
    return pl.kernel(
        _deg_body,
        out_type=jax.ShapeDtypeStruct((NC, NPAD), jnp.float32),
        mesh=_mesh,
        scratch_types=[
            pltpu.VMEM((CPT, CHUNK), jnp.int32),
            pltpu.VMEM((1, CHUNK), jnp.float32),
            pltpu.VMEM((STRIPE,), jnp.float32),
            pltpu.VMEM_SHARED((NPAD,), jnp.float32),
            pltpu.SemaphoreType.DMA,
        ],
    )(dst2d)


IDXG = 8


def _prop_body(g_hbm, src_hbm, dst_hbm, out_hbm, srcv, dstv, rows, rows2, acc,
               sem, sem2, sems0, sems1):
    c = lax.axis_index("c")
    s = lax.axis_index("s")
    w = c * NS + s
    d = rows.shape[1]
    _fill_vmem(rows, CHUNK, d, 0.0)

    def zs(i, _):
        pltpu.sync_copy(rows, acc.at[pl.ds(s * STRIPE + i * CHUNK, CHUNK)])
        return 0

    lax.fori_loop(0, STRIPE // CHUNK, zs, 0)
    plsc.subcore_barrier()

    def group(jg, _):
        base = w * CPT + jg * IDXG
        pltpu.sync_copy(src_hbm.at[pl.ds(base, IDXG)], srcv)
        pltpu.sync_copy(dst_hbm.at[pl.ds(base, IDXG)], dstv)
        bufs = (rows, rows2)
        gsems = (sem, sem2)
        ssems = (sems0, sems1)
        gd = [None, None]
        sd = [None, None]
        gd[0] = pltpu.async_copy(g_hbm.at[srcv.at[0]], bufs[0], gsems[0])
        for jj in range(IDXG):
            b = jj % 2
            nb = (jj + 1) % 2
            if jj + 1 < IDXG:
                if sd[nb] is not None:
                    sd[nb].wait()
                gd[nb] = pltpu.async_copy(g_hbm.at[srcv.at[jj + 1]],
                                          bufs[nb], gsems[nb])
            gd[b].wait()
            sd[b] = pltpu.async_copy(bufs[b], acc.at[dstv.at[jj]], ssems[b],
                                     add=True)
        for b in range(2):
            if sd[b] is not None:
                sd[b].wait()
        return 0

    lax.fori_loop(0, CPT // IDXG, group, 0)
    plsc.subcore_barrier()
    pltpu.sync_copy(acc.at[pl.ds(s * STRIPE, STRIPE)],
                    out_hbm.at[c, pl.ds(s * STRIPE, STRIPE)])


def _propagate(g, src2d, dst2d, d):
    return pl.kernel(
        _prop_body,
        out_type=jax.ShapeDtypeStruct((NC, NPAD, d), jnp.float32),
        mesh=_mesh,
        scratch_types=[
            pltpu.VMEM((IDXG, CHUNK), jnp.int32),
            pltpu.VMEM((IDXG, CHUNK), jnp.int32),
            pltpu.VMEM((CHUNK, d), jnp.float32),
            pltpu.VMEM((CHUNK, d), jnp.float32),
            pltpu.VMEM_SHARED((NPAD, d), jnp.float32),
            pltpu.SemaphoreType.DMA,
            pltpu.SemaphoreType.DMA,
            pltpu.SemaphoreType.DMA,
            pltpu.SemaphoreType.DMA,
        ],
        compiler_params=pltpu.CompilerParams(use_tc_tiling_on_sc=False),
    )(g, src2d, dst2d)


_MBLK = 1024
_GRID = NPAD // _MBLK


def _col_spec():
    return pl.BlockSpec((_MBLK, 1), lambda i: (i, 0))


def _mat_spec(d):
    return pl.BlockSpec((_MBLK, d), lambda i: (i, 0))


def _full_spec(r, c):
    return pl.BlockSpec((r, c), lambda i: (0, 0))


def _tc1_body(p0, p1, m, x, w1, g1, dv):
    deg = p0[...] + p1[...] + m[...]
    dinv = jnp.where(deg > 0, lax.rsqrt(deg), 0.0)
    g1[...] = dinv * jnp.dot(x[...], w1[...],
                             preferred_element_type=jnp.float32)
    dv[...] = dinv


def _tc1(p0, p1, m, x, w1):
    return pl.pallas_call(
        _tc1_body,
        grid=(_GRID,),
        in_specs=[_col_spec(), _col_spec(), _col_spec(), _mat_spec(128),
                  _full_spec(128, 128)],
        out_specs=[_mat_spec(128), _col_spec()],
        out_shape=[jax.ShapeDtypeStruct((NPAD, 128), jnp.float32),
                   jax.ShapeDtypeStruct((NPAD, 1), jnp.float32)],
    )(p0, p1, m, x, w1)


def _tc2_body(q0, q1, g1, dv, b1, w2, g2):
    dinv = dv[...]
    h1 = jnp.maximum(dinv * (q0[...] + q1[...] + g1[...]) + b1[...], 0.0)
    g2[...] = dinv * jnp.dot(h1, w2[...], preferred_element_type=jnp.float32)


def _tc2(q0, q1, g1, dv, b1, w2):
    return pl.pallas_call(
        _tc2_body,
        grid=(_GRID,),
        in_specs=[_mat_spec(128), _mat_spec(128), _mat_spec(128), _col_spec(),
                  _full_spec(1, 128), _full_spec(128, 64)],
        out_specs=_mat_spec(64),
        out_shape=jax.ShapeDtypeStruct((NPAD, 64), jnp.float32),
    )(q0, q1, g1, dv, b1, w2)


def _tc3_body(r0, r1, g2, dv, b2, out):
    out[...] = jnp.maximum(
        dv[...] * (r0[...] + r1[...] + g2[...]) + b2[...], 0.0)


def _tc3(r0, r1, g2, dv, b2):
    return pl.pallas_call(
        _tc3_body,
        grid=(_GRID,),
        in_specs=[_mat_spec(64), _mat_spec(64), _mat_spec(64), _col_spec(),
                  _full_spec(1, 64)],
        out_specs=_mat_spec(64),
        out_shape=jax.ShapeDtypeStruct((NPAD, 64), jnp.float32),
    )(r0, r1, g2, dv, b2)


@jax.jit
def kernel(x, edge_index, W1, b1, W2, b2):
    src = edge_index[0].astype(jnp.int32)
    dst = edge_index[1].astype(jnp.int32)
    pad = EPAD - E
    src2d = jnp.concatenate(
        [src, jnp.full((pad,), PAD_SRC, jnp.int32)]).reshape(ROWS2D, CHUNK)
    dst2d = jnp.concatenate(
        [dst, jnp.full((pad,), PAD_DST, jnp.int32)]).reshape(ROWS2D, CHUNK)
    xp = jnp.pad(x, ((0, NPAD - N), (0, 0)))
    maskc = (jnp.arange(NPAD) < N).astype(jnp.float32).reshape(NPAD, 1)

    degp = _degree_kernel(dst2d)
    p0 = degp[0].reshape(NPAD, 1)
    p1 = degp[1].reshape(NPAD, 1)
    g1, dv = _tc1(p0, p1, maskc, xp, W1)

    s1 = _propagate(g1, src2d, dst2d, 128)
    g2 = _tc2(s1[0], s1[1], g1, dv, b1.reshape(1, 128), W2)

    s2 = _propagate(g2, src2d, dst2d, 64)
    out = _tc3(s2[0], s2[1], g2, dv, b2.reshape(1, 64))
    return out[:N]


# speedup vs baseline: 21.8901x; 1.9560x over previous
import jax
import jax.numpy as jnp
from jax import lax
from jax.experimental import pallas as pl
from jax.experimental.pallas import tpu as pltpu
from jax.experimental.pallas import tpu_sc as plsc

N = 10000
NPAD = 10240
E = 320000
NC, NS = 2, 16
NW = NC * NS
CHUNK = 128
CPT = 80
EPW = CPT * CHUNK
EPAD = EPW * NW
ROWS2D = EPAD // CHUNK
STRIPE = NPAD // NS
PAD_SRC = N
PAD_DST = N + 100

_mesh = plsc.VectorSubcoreMesh(core_axis_name="c", subcore_axis_name="s")


def _fill_vmem(ref, rows, width, value):
    kpr = width // 16

    def body(i, _):
        r = i // kpr
        k = i % kpr
        ref[r, pl.ds(k * 16, 16)] = jnp.full((16,), value, jnp.float32)
        return 0

    lax.fori_loop(0, rows * kpr, body, 0)


def _fill_vmem1d(ref, n, value):
    def body(i, _):
        ref[pl.ds(i * 16, 16)] = jnp.full((16,), value, jnp.float32)
        return 0

    lax.fori_loop(0, n // 16, body, 0)


def _deg_body(dst_hbm, out_hbm, dstv, onesv, zrow, acc, sem):
    del sem
    c = lax.axis_index("c")
    s = lax.axis_index("s")
    w = c * NS + s
    _fill_vmem(onesv, 1, CHUNK, 1.0)
    _fill_vmem1d(zrow, STRIPE, 0.0)
    pltpu.sync_copy(zrow, acc.at[pl.ds(s * STRIPE, STRIPE)])
    plsc.subcore_barrier()
    pltpu.sync_copy(dst_hbm.at[pl.ds(w * CPT, CPT)], dstv)

    def body(j, _):
        pltpu.sync_copy(onesv.at[0], acc.at[dstv.at[j]], add=True)
        return 0

    lax.fori_loop(0, CPT, body, 0)
    plsc.subcore_barrier()
    pltpu.sync_copy(acc.at[pl.ds(s * STRIPE, STRIPE)],
                    out_hbm.at[c, pl.ds(s * STRIPE, STRIPE)])


def _degree_kernel(dst2d):
    return pl.kernel(
        _deg_body,
        out_type=jax.ShapeDtypeStruct((NC, NPAD), jnp.float32),
        mesh=_mesh,
        scratch_types=[
            pltpu.VMEM((CPT, CHUNK), jnp.int32),
            pltpu.VMEM((1, CHUNK), jnp.float32),
            pltpu.VMEM((STRIPE,), jnp.float32),
            pltpu.VMEM_SHARED((NPAD,), jnp.float32),
            pltpu.SemaphoreType.DMA,
        ],
    )(dst2d)


IDXG = 8
NBUF = 2
CPT2 = ROWS2D // NS


def _prop_body(g_hbm, src_hbm, dst_hbm, out_hbm, srcv, dstv, bufs_and_sems,
               gsh, acc):
    c = lax.axis_index("c")
    s = lax.axis_index("s")
    bufs = bufs_and_sems[:NBUF]
    gsems = bufs_and_sems[NBUF:2 * NBUF]
    ssems = bufs_and_sems[2 * NBUF:]
    d2 = bufs[0].shape[1]
    pltpu.sync_copy(g_hbm.at[c, pl.ds(s * STRIPE, STRIPE)],
                    gsh.at[pl.ds(s * STRIPE, STRIPE)])
    _fill_vmem(bufs[0], CHUNK, d2, 0.0)

    def zs(i, _):
        pltpu.sync_copy(bufs[0], acc.at[pl.ds(s * STRIPE + i * CHUNK, CHUNK)])
        return 0

    lax.fori_loop(0, STRIPE // CHUNK, zs, 0)
    plsc.subcore_barrier()

    def group(jg, _):
        base = s * CPT2 + jg * IDXG
        pltpu.sync_copy(src_hbm.at[pl.ds(base, IDXG)], srcv)
        pltpu.sync_copy(dst_hbm.at[pl.ds(base, IDXG)], dstv)
        gd = [None] * NBUF
        sd = [None] * NBUF
        gd[0] = pltpu.async_copy(gsh.at[srcv.at[0]], bufs[0], gsems[0])
        for jj in range(IDXG):
            b = jj % NBUF
            if jj + 1 < IDXG:
                nb = (jj + 1) % NBUF
                if sd[nb] is not None:
                    sd[nb].wait()
                gd[nb] = pltpu.async_copy(gsh.at[srcv.at[jj + 1]],
                                          bufs[nb], gsems[nb])
            gd[b].wait()
            sd[b] = pltpu.async_copy(bufs[b], acc.at[dstv.at[jj]], ssems[b],
                                     add=True)
        for b in range(NBUF):
            if sd[b] is not None:
                sd[b].wait()
        return 0

    lax.fori_loop(0, CPT2 // IDXG, group, 0)
    plsc.subcore_barrier()
    pltpu.sync_copy(acc.at[pl.ds(s * STRIPE, STRIPE)],
                    out_hbm.at[c, pl.ds(s * STRIPE, STRIPE)])


def _propagate(gh, src2d, dst2d, d2):
    return pl.kernel(
        _prop_body,
        out_type=jax.ShapeDtypeStruct((NC, NPAD, d2), jnp.float32),
        mesh=_mesh,
        scratch_types=[
            pltpu.VMEM((IDXG, CHUNK), jnp.int32),
            pltpu.VMEM((IDXG, CHUNK), jnp.int32),
            [pltpu.VMEM((CHUNK, d2), jnp.float32) for _ in range(NBUF)]
            + [pltpu.SemaphoreType.DMA for _ in range(2 * NBUF)],
            pltpu.VMEM_SHARED((NPAD, d2), jnp.float32),
            pltpu.VMEM_SHARED((NPAD, d2), jnp.float32),
        ],
        compiler_params=pltpu.CompilerParams(use_tc_tiling_on_sc=False),
    )(gh, src2d, dst2d)


_MBLK = 1024
_GRID = NPAD // _MBLK


def _col_spec():
    return pl.BlockSpec((_MBLK, 1), lambda i: (i, 0))


def _mat_spec(d):
    return pl.BlockSpec((_MBLK, d), lambda i: (i, 0))


def _full_spec(r, c):
    return pl.BlockSpec((r, c), lambda i: (0, 0))


def _tc1_body(p0, p1, m, x, w1, g1, dv):
    deg = p0[...] + p1[...] + m[...]
    dinv = jnp.where(deg > 0, lax.rsqrt(deg), 0.0)
    g1[...] = dinv * jnp.dot(x[...], w1[...],
                             preferred_element_type=jnp.float32)
    dv[...] = dinv


def _tc1(p0, p1, m, x, w1):
    return pl.pallas_call(
        _tc1_body,
        grid=(_GRID,),
        in_specs=[_col_spec(), _col_spec(), _col_spec(), _mat_spec(128),
                  _full_spec(128, 128)],
        out_specs=[_mat_spec(128), _col_spec()],
        out_shape=[jax.ShapeDtypeStruct((NPAD, 128), jnp.float32),
                   jax.ShapeDtypeStruct((NPAD, 1), jnp.float32)],
    )(p0, p1, m, x, w1)


def _tc2_body(q, g1, dv, b1, w2, g2):
    dinv = dv[...]
    h1 = jnp.maximum(dinv * (q[...] + g1[...]) + b1[...], 0.0)
    g2[...] = dinv * jnp.dot(h1, w2[...], preferred_element_type=jnp.float32)


def _tc2(q, g1, dv, b1, w2):
    return pl.pallas_call(
        _tc2_body,
        grid=(_GRID,),
        in_specs=[_mat_spec(128), _mat_spec(128), _col_spec(),
                  _full_spec(1, 128), _full_spec(128, 64)],
        out_specs=_mat_spec(64),
        out_shape=jax.ShapeDtypeStruct((NPAD, 64), jnp.float32),
    )(q, g1, dv, b1, w2)


def _tc3_body(r, g2, dv, b2, out):
    out[...] = jnp.maximum(dv[...] * (r[...] + g2[...]) + b2[...], 0.0)


def _tc3(r, g2, dv, b2):
    return pl.pallas_call(
        _tc3_body,
        grid=(_GRID,),
        in_specs=[_mat_spec(64), _mat_spec(64), _col_spec(),
                  _full_spec(1, 64)],
        out_specs=_mat_spec(64),
        out_shape=jax.ShapeDtypeStruct((NPAD, 64), jnp.float32),
    )(r, g2, dv, b2)


@jax.jit
def kernel(x, edge_index, W1, b1, W2, b2):
    src = edge_index[0].astype(jnp.int32)
    dst = edge_index[1].astype(jnp.int32)
    pad = EPAD - E
    src2d = jnp.concatenate(
        [src, jnp.full((pad,), PAD_SRC, jnp.int32)]).reshape(ROWS2D, CHUNK)
    dst2d = jnp.concatenate(
        [dst, jnp.full((pad,), PAD_DST, jnp.int32)]).reshape(ROWS2D, CHUNK)
    xp = jnp.pad(x, ((0, NPAD - N), (0, 0)))
    maskc = (jnp.arange(NPAD) < N).astype(jnp.float32).reshape(NPAD, 1)

    degp = _degree_kernel(dst2d)
    p0 = degp[0].reshape(NPAD, 1)
    p1 = degp[1].reshape(NPAD, 1)
    g1, dv = _tc1(p0, p1, maskc, xp, W1)

    g1h = jnp.stack([g1[:, :64], g1[:, 64:]])
    s1h = _propagate(g1h, src2d, dst2d, 64)
    s1 = jnp.concatenate([s1h[0], s1h[1]], axis=1)
    g2 = _tc2(s1, g1, dv, b1.reshape(1, 128), W2)

    g2h = jnp.stack([g2[:, :32], g2[:, 32:]])
    s2h = _propagate(g2h, src2d, dst2d, 32)
    s2 = jnp.concatenate([s2h[0], s2h[1]], axis=1)
    out = _tc3(s2, g2, dv, b2.reshape(1, 64))
    return out[:N]


# speedup vs baseline: 25.8572x; 1.1812x over previous
import jax
import jax.numpy as jnp
from jax import lax
from jax.experimental import pallas as pl
from jax.experimental.pallas import tpu as pltpu
from jax.experimental.pallas import tpu_sc as plsc

N = 10000
NPAD = 10240
E = 320000
NC, NS = 2, 16
NW = NC * NS
CHUNK = 128
CPT = 80
EPW = CPT * CHUNK
EPAD = EPW * NW
ROWS2D = EPAD // CHUNK
STRIPE = NPAD // NS
PAD_SRC = N
PAD_DST = N + 100

_mesh = plsc.VectorSubcoreMesh(core_axis_name="c", subcore_axis_name="s")


def _fill_vmem(ref, rows, width, value):
    kpr = width // 16

    def body(i, _):
        r = i // kpr
        k = i % kpr
        ref[r, pl.ds(k * 16, 16)] = jnp.full((16,), value, jnp.float32)
        return 0

    lax.fori_loop(0, rows * kpr, body, 0)


def _fill_vmem1d(ref, n, value):
    def body(i, _):
        ref[pl.ds(i * 16, 16)] = jnp.full((16,), value, jnp.float32)
        return 0

    lax.fori_loop(0, n // 16, body, 0)


def _deg_body(dst_hbm, out_hbm, dstv, onesv, zrow, acc, sem):
    del sem
    c = lax.axis_index("c")
    s = lax.axis_index("s")
    w = c * NS + s
    _fill_vmem(onesv, 1, CHUNK, 1.0)
    _fill_vmem1d(zrow, STRIPE, 0.0)
    pltpu.sync_copy(zrow, acc.at[pl.ds(s * STRIPE, STRIPE)])
    plsc.subcore_barrier()
    pltpu.sync_copy(dst_hbm.at[pl.ds(w * CPT, CPT)], dstv)

    def body(j, _):
        pltpu.sync_copy(onesv.at[0], acc.at[dstv.at[j]], add=True)
        return 0

    lax.fori_loop(0, CPT, body, 0)
    plsc.subcore_barrier()
    pltpu.sync_copy(acc.at[pl.ds(s * STRIPE, STRIPE)],
                    out_hbm.at[c, pl.ds(s * STRIPE, STRIPE)])


def _degree_kernel(dst2d):
    return pl.kernel(
        _deg_body,
        out_type=jax.ShapeDtypeStruct((NC, NPAD), jnp.float32),
        mesh=_mesh,
        scratch_types=[
            pltpu.VMEM((CPT, CHUNK), jnp.int32),
            pltpu.VMEM((1, CHUNK), jnp.float32),
            pltpu.VMEM((STRIPE,), jnp.float32),
            pltpu.VMEM_SHARED((NPAD,), jnp.float32),
            pltpu.SemaphoreType.DMA,
        ],
    )(dst2d)


IDXG = 16
NBUF = 4
CPT2 = ROWS2D // NS


def _prop_body(g_hbm, src_hbm, dst_hbm, out_hbm, srcv, dstv, bufs_and_sems,
               gsh, acc):
    c = lax.axis_index("c")
    s = lax.axis_index("s")
    bufs = bufs_and_sems[:NBUF]
    gsems = bufs_and_sems[NBUF:2 * NBUF]
    ssems = bufs_and_sems[2 * NBUF:]
    d2 = bufs[0].shape[1]
    pltpu.sync_copy(g_hbm.at[c, pl.ds(s * STRIPE, STRIPE)],
                    gsh.at[pl.ds(s * STRIPE, STRIPE)])
    _fill_vmem(bufs[0], CHUNK, d2, 0.0)

    def zs(i, _):
        pltpu.sync_copy(bufs[0], acc.at[pl.ds(s * STRIPE + i * CHUNK, CHUNK)])
        return 0

    lax.fori_loop(0, STRIPE // CHUNK, zs, 0)
    plsc.subcore_barrier()

    def group(jg, _):
        base = s * CPT2 + jg * IDXG
        pltpu.sync_copy(src_hbm.at[pl.ds(base, IDXG)], srcv)
        pltpu.sync_copy(dst_hbm.at[pl.ds(base, IDXG)], dstv)
        gd = [None] * NBUF
        sd = [None] * NBUF
        gd[0] = pltpu.async_copy(gsh.at[srcv.at[0]], bufs[0], gsems[0])
        for jj in range(IDXG):
            b = jj % NBUF
            if jj + 1 < IDXG:
                nb = (jj + 1) % NBUF
                if sd[nb] is not None:
                    sd[nb].wait()
                gd[nb] = pltpu.async_copy(gsh.at[srcv.at[jj + 1]],
                                          bufs[nb], gsems[nb])
            gd[b].wait()
            sd[b] = pltpu.async_copy(bufs[b], acc.at[dstv.at[jj]], ssems[b],
                                     add=True)
        for b in range(NBUF):
            if sd[b] is not None:
                sd[b].wait()
        return 0

    lax.fori_loop(0, CPT2 // IDXG, group, 0)
    plsc.subcore_barrier()
    pltpu.sync_copy(acc.at[pl.ds(s * STRIPE, STRIPE)],
                    out_hbm.at[c, pl.ds(s * STRIPE, STRIPE)])


def _propagate(gh, src2d, dst2d, d2):
    return pl.kernel(
        _prop_body,
        out_type=jax.ShapeDtypeStruct((NC, NPAD, d2), jnp.float32),
        mesh=_mesh,
        scratch_types=[
            pltpu.VMEM((IDXG, CHUNK), jnp.int32),
            pltpu.VMEM((IDXG, CHUNK), jnp.int32),
            [pltpu.VMEM((CHUNK, d2), jnp.float32) for _ in range(NBUF)]
            + [pltpu.SemaphoreType.DMA for _ in range(2 * NBUF)],
            pltpu.VMEM_SHARED((NPAD, d2), jnp.float32),
            pltpu.VMEM_SHARED((NPAD, d2), jnp.float32),
        ],
        compiler_params=pltpu.CompilerParams(use_tc_tiling_on_sc=False),
    )(gh, src2d, dst2d)


_MBLK = 1024
_GRID = NPAD // _MBLK


def _col_spec():
    return pl.BlockSpec((_MBLK, 1), lambda i: (i, 0))


def _mat_spec(d):
    return pl.BlockSpec((_MBLK, d), lambda i: (i, 0))


def _full_spec(r, c):
    return pl.BlockSpec((r, c), lambda i: (0, 0))


def _tc1_body(p0, p1, m, x, w1, g1, dv):
    deg = p0[...] + p1[...] + m[...]
    dinv = jnp.where(deg > 0, lax.rsqrt(deg), 0.0)
    g1[...] = dinv * jnp.dot(x[...], w1[...],
                             preferred_element_type=jnp.float32)
    dv[...] = dinv


def _tc1(p0, p1, m, x, w1):
    return pl.pallas_call(
        _tc1_body,
        grid=(_GRID,),
        in_specs=[_col_spec(), _col_spec(), _col_spec(), _mat_spec(128),
                  _full_spec(128, 128)],
        out_specs=[_mat_spec(128), _col_spec()],
        out_shape=[jax.ShapeDtypeStruct((NPAD, 128), jnp.float32),
                   jax.ShapeDtypeStruct((NPAD, 1), jnp.float32)],
    )(p0, p1, m, x, w1)


def _tc2_body(q, g1, dv, b1, w2, g2):
    dinv = dv[...]
    h1 = jnp.maximum(dinv * (q[...] + g1[...]) + b1[...], 0.0)
    g2[...] = dinv * jnp.dot(h1, w2[...], preferred_element_type=jnp.float32)


def _tc2(q, g1, dv, b1, w2):
    return pl.pallas_call(
        _tc2_body,
        grid=(_GRID,),
        in_specs=[_mat_spec(128), _mat_spec(128), _col_spec(),
                  _full_spec(1, 128), _full_spec(128, 64)],
        out_specs=_mat_spec(64),
        out_shape=jax.ShapeDtypeStruct((NPAD, 64), jnp.float32),
    )(q, g1, dv, b1, w2)


def _tc3_body(r, g2, dv, b2, out):
    out[...] = jnp.maximum(dv[...] * (r[...] + g2[...]) + b2[...], 0.0)


def _tc3(r, g2, dv, b2):
    return pl.pallas_call(
        _tc3_body,
        grid=(_GRID,),
        in_specs=[_mat_spec(64), _mat_spec(64), _col_spec(),
                  _full_spec(1, 64)],
        out_specs=_mat_spec(64),
        out_shape=jax.ShapeDtypeStruct((NPAD, 64), jnp.float32),
    )(r, g2, dv, b2)


@jax.jit
def kernel(x, edge_index, W1, b1, W2, b2):
    src = edge_index[0].astype(jnp.int32)
    dst = edge_index[1].astype(jnp.int32)
    pad = EPAD - E
    src2d = jnp.concatenate(
        [src, jnp.full((pad,), PAD_SRC, jnp.int32)]).reshape(ROWS2D, CHUNK)
    dst2d = jnp.concatenate(
        [dst, jnp.full((pad,), PAD_DST, jnp.int32)]).reshape(ROWS2D, CHUNK)
    xp = jnp.pad(x, ((0, NPAD - N), (0, 0)))
    maskc = (jnp.arange(NPAD) < N).astype(jnp.float32).reshape(NPAD, 1)

    degp = _degree_kernel(dst2d)
    p0 = degp[0].reshape(NPAD, 1)
    p1 = degp[1].reshape(NPAD, 1)
    g1, dv = _tc1(p0, p1, maskc, xp, W1)

    g1h = jnp.stack([g1[:, :64], g1[:, 64:]])
    s1h = _propagate(g1h, src2d, dst2d, 64)
    s1 = jnp.concatenate([s1h[0], s1h[1]], axis=1)
    g2 = _tc2(s1, g1, dv, b1.reshape(1, 128), W2)

    g2h = jnp.stack([g2[:, :32], g2[:, 32:]])
    s2h = _propagate(g2h, src2d, dst2d, 32)
    s2 = jnp.concatenate([s2h[0], s2h[1]], axis=1)
    out = _tc3(s2, g2, dv, b2.reshape(1, 64))
    return out[:N]


# speedup vs baseline: 29.9799x; 1.1594x over previous
import jax
import jax.numpy as jnp
from jax import lax
from jax.experimental import pallas as pl
from jax.experimental.pallas import tpu as pltpu
from jax.experimental.pallas import tpu_sc as plsc

N = 10000
NPAD = 10240
E = 320000
NC, NS = 2, 16
NW = NC * NS
CHUNK = 128
CPT = 80
EPW = CPT * CHUNK
EPAD = EPW * NW
ROWS2D = EPAD // CHUNK
STRIPE = NPAD // NS
PAD_SRC = N
PAD_DST = N + 100

_mesh = plsc.VectorSubcoreMesh(core_axis_name="c", subcore_axis_name="s")


def _fill_vmem(ref, rows, width, value):
    kpr = width // 16

    def body(i, _):
        r = i // kpr
        k = i % kpr
        ref[r, pl.ds(k * 16, 16)] = jnp.full((16,), value, jnp.float32)
        return 0

    lax.fori_loop(0, rows * kpr, body, 0)


def _fill_vmem1d(ref, n, value):
    def body(i, _):
        ref[pl.ds(i * 16, 16)] = jnp.full((16,), value, jnp.float32)
        return 0

    lax.fori_loop(0, n // 16, body, 0)


def _deg_body(dst_hbm, out_hbm, dstv, onesv, zrow, acc, sem):
    del sem
    c = lax.axis_index("c")
    s = lax.axis_index("s")
    w = c * NS + s
    _fill_vmem(onesv, 1, CHUNK, 1.0)
    _fill_vmem1d(zrow, STRIPE, 0.0)
    pltpu.sync_copy(zrow, acc.at[pl.ds(s * STRIPE, STRIPE)])
    plsc.subcore_barrier()
    pltpu.sync_copy(dst_hbm.at[pl.ds(w * CPT, CPT)], dstv)

    def body(j, _):
        pltpu.sync_copy(onesv.at[0], acc.at[dstv.at[j]], add=True)
        return 0

    lax.fori_loop(0, CPT, body, 0)
    plsc.subcore_barrier()
    pltpu.sync_copy(acc.at[pl.ds(s * STRIPE, STRIPE)],
                    out_hbm.at[c, pl.ds(s * STRIPE, STRIPE)])


def _degree_kernel(dst2d):
    return pl.kernel(
        _deg_body,
        out_type=jax.ShapeDtypeStruct((NC, NPAD), jnp.float32),
        mesh=_mesh,
        scratch_types=[
            pltpu.VMEM((CPT, CHUNK), jnp.int32),
            pltpu.VMEM((1, CHUNK), jnp.float32),
            pltpu.VMEM((STRIPE,), jnp.float32),
            pltpu.VMEM_SHARED((NPAD,), jnp.float32),
            pltpu.SemaphoreType.DMA,
        ],
    )(dst2d)


IDXG = 16
NBUF = 4
CPT2 = ROWS2D // NS


def _prop_body(g_hbm, src_hbm, dst_hbm, out_hbm, srcv, dstv, bufs_and_sems,
               gsh, acc):
    c = lax.axis_index("c")
    s = lax.axis_index("s")
    bufs = bufs_and_sems[:NBUF]
    gsems = bufs_and_sems[NBUF:2 * NBUF]
    ssems = bufs_and_sems[2 * NBUF:]
    d2 = bufs[0].shape[1]
    pltpu.sync_copy(g_hbm.at[pl.ds(s * STRIPE, STRIPE), pl.ds(c * d2, d2)],
                    gsh.at[pl.ds(s * STRIPE, STRIPE)])
    _fill_vmem(bufs[0], CHUNK, d2, 0.0)

    def zs(i, _):
        pltpu.sync_copy(bufs[0], acc.at[pl.ds(s * STRIPE + i * CHUNK, CHUNK)])
        return 0

    lax.fori_loop(0, STRIPE // CHUNK, zs, 0)
    plsc.subcore_barrier()

    def group(jg, _):
        base = s * CPT2 + jg * IDXG
        pltpu.sync_copy(src_hbm.at[pl.ds(base, IDXG)], srcv)
        pltpu.sync_copy(dst_hbm.at[pl.ds(base, IDXG)], dstv)
        gd = [None] * NBUF
        sd = [None] * NBUF
        gd[0] = pltpu.async_copy(gsh.at[srcv.at[0]], bufs[0], gsems[0])
        for jj in range(IDXG):
            b = jj % NBUF
            if jj + 1 < IDXG:
                nb = (jj + 1) % NBUF
                if sd[nb] is not None:
                    sd[nb].wait()
                gd[nb] = pltpu.async_copy(gsh.at[srcv.at[jj + 1]],
                                          bufs[nb], gsems[nb])
            gd[b].wait()
            sd[b] = pltpu.async_copy(bufs[b], acc.at[dstv.at[jj]], ssems[b],
                                     add=True)
        for b in range(NBUF):
            if sd[b] is not None:
                sd[b].wait()
        return 0

    lax.fori_loop(0, CPT2 // IDXG, group, 0)
    plsc.subcore_barrier()
    pltpu.sync_copy(acc.at[pl.ds(s * STRIPE, STRIPE)],
                    out_hbm.at[pl.ds(s * STRIPE, STRIPE), pl.ds(c * d2, d2)])


def _propagate(gh, src2d, dst2d, d2):
    return pl.kernel(
        _prop_body,
        out_type=jax.ShapeDtypeStruct((NPAD, 2 * d2), jnp.float32),
        mesh=_mesh,
        scratch_types=[
            pltpu.VMEM((IDXG, CHUNK), jnp.int32),
            pltpu.VMEM((IDXG, CHUNK), jnp.int32),
            [pltpu.VMEM((CHUNK, d2), jnp.float32) for _ in range(NBUF)]
            + [pltpu.SemaphoreType.DMA for _ in range(2 * NBUF)],
            pltpu.VMEM_SHARED((NPAD, d2), jnp.float32),
            pltpu.VMEM_SHARED((NPAD, d2), jnp.float32),
        ],
        compiler_params=pltpu.CompilerParams(use_tc_tiling_on_sc=False),
    )(gh, src2d, dst2d)


_MBLK = 1024
_GRID = NPAD // _MBLK


def _col_spec():
    return pl.BlockSpec((_MBLK, 1), lambda i: (i, 0))


def _mat_spec(d):
    return pl.BlockSpec((_MBLK, d), lambda i: (i, 0))


def _full_spec(r, c):
    return pl.BlockSpec((r, c), lambda i: (0, 0))


def _tc1_body(p0, p1, m, x, w1, g1, dv):
    deg = p0[...] + p1[...] + m[...]
    dinv = jnp.where(deg > 0, lax.rsqrt(deg), 0.0)
    g1[...] = dinv * jnp.dot(x[...], w1[...],
                             preferred_element_type=jnp.float32)
    dv[...] = dinv


def _tc1(p0, p1, m, x, w1):
    return pl.pallas_call(
        _tc1_body,
        grid=(_GRID,),
        in_specs=[_col_spec(), _col_spec(), _col_spec(), _mat_spec(128),
                  _full_spec(128, 128)],
        out_specs=[_mat_spec(128), _col_spec()],
        out_shape=[jax.ShapeDtypeStruct((NPAD, 128), jnp.float32),
                   jax.ShapeDtypeStruct((NPAD, 1), jnp.float32)],
    )(p0, p1, m, x, w1)


def _tc2_body(q, g1, dv, b1, w2, g2):
    dinv = dv[...]
    h1 = jnp.maximum(dinv * (q[...] + g1[...]) + b1[...], 0.0)
    g2[...] = dinv * jnp.dot(h1, w2[...], preferred_element_type=jnp.float32)


def _tc2(q, g1, dv, b1, w2):
    return pl.pallas_call(
        _tc2_body,
        grid=(_GRID,),
        in_specs=[_mat_spec(128), _mat_spec(128), _col_spec(),
                  _full_spec(1, 128), _full_spec(128, 64)],
        out_specs=_mat_spec(64),
        out_shape=jax.ShapeDtypeStruct((NPAD, 64), jnp.float32),
    )(q, g1, dv, b1, w2)


def _tc3_body(r, g2, dv, b2, out):
    out[...] = jnp.maximum(dv[...] * (r[...] + g2[...]) + b2[...], 0.0)


def _tc3(r, g2, dv, b2):
    return pl.pallas_call(
        _tc3_body,
        grid=(_GRID,),
        in_specs=[_mat_spec(64), _mat_spec(64), _col_spec(),
                  _full_spec(1, 64)],
        out_specs=_mat_spec(64),
        out_shape=jax.ShapeDtypeStruct((NPAD, 64), jnp.float32),
    )(r, g2, dv, b2)


@jax.jit
def kernel(x, edge_index, W1, b1, W2, b2):
    src = edge_index[0].astype(jnp.int32)
    dst = edge_index[1].astype(jnp.int32)
    pad = EPAD - E
    src2d = jnp.concatenate(
        [src, jnp.full((pad,), PAD_SRC, jnp.int32)]).reshape(ROWS2D, CHUNK)
    dst2d = jnp.concatenate(
        [dst, jnp.full((pad,), PAD_DST, jnp.int32)]).reshape(ROWS2D, CHUNK)
    xp = jnp.pad(x, ((0, NPAD - N), (0, 0)))
    maskc = (jnp.arange(NPAD) < N).astype(jnp.float32).reshape(NPAD, 1)

    degp = _degree_kernel(dst2d)
    p0 = degp[0].reshape(NPAD, 1)
    p1 = degp[1].reshape(NPAD, 1)
    g1, dv = _tc1(p0, p1, maskc, xp, W1)

    s1 = _propagate(g1, src2d, dst2d, 64)
    g2 = _tc2(s1, g1, dv, b1.reshape(1, 128), W2)

    s2 = _propagate(g2, src2d, dst2d, 32)
    out = _tc3(s2, g2, dv, b2.reshape(1, 64))
    return out[:N]


# speedup vs baseline: 37.1320x; 1.2386x over previous
import jax
import jax.numpy as jnp
from jax import lax
from jax.experimental import pallas as pl
from jax.experimental.pallas import tpu as pltpu
from jax.experimental.pallas import tpu_sc as plsc

N = 10000
NPAD = 10240
E = 320000
NC, NS = 2, 16
NW = NC * NS
CHUNK = 128
CPT = 80
EPW = CPT * CHUNK
EPAD = EPW * NW
ROWS2D = EPAD // CHUNK
STRIPE = NPAD // NS
PAD_SRC = N
PAD_DST = N + 100

_mesh = plsc.VectorSubcoreMesh(core_axis_name="c", subcore_axis_name="s")


def _fill_vmem(ref, rows, width, value):
    lanes = 32 if ref.dtype == jnp.bfloat16 else 16
    kpr = width // lanes

    def body(i, _):
        r = i // kpr
        k = i % kpr
        ref[r, pl.ds(k * lanes, lanes)] = jnp.full((lanes,), value, ref.dtype)
        return 0

    lax.fori_loop(0, rows * kpr, body, 0)


def _fill_vmem1d(ref, n, value):
    def body(i, _):
        ref[pl.ds(i * 16, 16)] = jnp.full((16,), value, jnp.float32)
        return 0

    lax.fori_loop(0, n // 16, body, 0)


def _deg_body(dst_hbm, out_hbm, dstv, onesv, zrow, acc, sem):
    del sem
    c = lax.axis_index("c")
    s = lax.axis_index("s")
    w = c * NS + s
    _fill_vmem(onesv, 1, CHUNK, 1.0)
    _fill_vmem1d(zrow, STRIPE, 0.0)
    pltpu.sync_copy(zrow, acc.at[pl.ds(s * STRIPE, STRIPE)])
    plsc.subcore_barrier()
    pltpu.sync_copy(dst_hbm.at[pl.ds(w * CPT, CPT)], dstv)

    def body(j, _):
        pltpu.sync_copy(onesv.at[0], acc.at[dstv.at[j]], add=True)
        return 0

    lax.fori_loop(0, CPT, body, 0)
    plsc.subcore_barrier()
    pltpu.sync_copy(acc.at[pl.ds(s * STRIPE, STRIPE)],
                    out_hbm.at[c, pl.ds(s * STRIPE, STRIPE)])


def _degree_kernel(dst2d):
    return pl.kernel(
        _deg_body,
        out_type=jax.ShapeDtypeStruct((NC, NPAD), jnp.float32),
        mesh=_mesh,
        scratch_types=[
            pltpu.VMEM((CPT, CHUNK), jnp.int32),
            pltpu.VMEM((1, CHUNK), jnp.float32),
            pltpu.VMEM((STRIPE,), jnp.float32),
            pltpu.VMEM_SHARED((NPAD,), jnp.float32),
            pltpu.SemaphoreType.DMA,
        ],
    )(dst2d)


IDXG = 16
NBUF = 4
CPT2 = ROWS2D // NS


def _prop_body(g_hbm, src_hbm, dst_hbm, out_hbm, srcv, dstv, bufs_and_sems,
               gsh, acc):
    c = lax.axis_index("c")
    s = lax.axis_index("s")
    bufs = bufs_and_sems[:NBUF]
    gsems = bufs_and_sems[NBUF:2 * NBUF]
    ssems = bufs_and_sems[2 * NBUF:]
    d2 = bufs[0].shape[1]
    pltpu.sync_copy(g_hbm.at[pl.ds(s * STRIPE, STRIPE), pl.ds(c * d2, d2)],
                    gsh.at[pl.ds(s * STRIPE, STRIPE)])
    _fill_vmem(bufs[0], CHUNK, d2, 0.0)

    def zs(i, _):
        pltpu.sync_copy(bufs[0], acc.at[pl.ds(s * STRIPE + i * CHUNK, CHUNK)])
        return 0

    lax.fori_loop(0, STRIPE // CHUNK, zs, 0)
    plsc.subcore_barrier()

    def group(jg, _):
        base = s * CPT2 + jg * IDXG
        pltpu.sync_copy(src_hbm.at[pl.ds(base, IDXG)], srcv)
        pltpu.sync_copy(dst_hbm.at[pl.ds(base, IDXG)], dstv)
        gd = [None] * NBUF
        sd = [None] * NBUF
        gd[0] = pltpu.async_copy(gsh.at[srcv.at[0]], bufs[0], gsems[0])
        for jj in range(IDXG):
            b = jj % NBUF
            if jj + 1 < IDXG:
                nb = (jj + 1) % NBUF
                if sd[nb] is not None:
                    sd[nb].wait()
                gd[nb] = pltpu.async_copy(gsh.at[srcv.at[jj + 1]],
                                          bufs[nb], gsems[nb])
            gd[b].wait()
            sd[b] = pltpu.async_copy(bufs[b], acc.at[dstv.at[jj]], ssems[b],
                                     add=True)
        for b in range(NBUF):
            if sd[b] is not None:
                sd[b].wait()
        return 0

    lax.fori_loop(0, CPT2 // IDXG, group, 0)
    plsc.subcore_barrier()
    pltpu.sync_copy(acc.at[pl.ds(s * STRIPE, STRIPE)],
                    out_hbm.at[pl.ds(s * STRIPE, STRIPE), pl.ds(c * d2, d2)])


def _propagate(gh, src2d, dst2d, d2):
    return pl.kernel(
        _prop_body,
        out_type=jax.ShapeDtypeStruct((NPAD, 2 * d2), jnp.bfloat16),
        mesh=_mesh,
        scratch_types=[
            pltpu.VMEM((IDXG, CHUNK), jnp.int32),
            pltpu.VMEM((IDXG, CHUNK), jnp.int32),
            [pltpu.VMEM((CHUNK, d2), jnp.bfloat16) for _ in range(NBUF)]
            + [pltpu.SemaphoreType.DMA for _ in range(2 * NBUF)],
            pltpu.VMEM_SHARED((NPAD, d2), jnp.bfloat16),
            pltpu.VMEM_SHARED((NPAD, d2), jnp.bfloat16),
        ],
        compiler_params=pltpu.CompilerParams(use_tc_tiling_on_sc=False),
    )(gh, src2d, dst2d)


_MBLK = 1024
_GRID = NPAD // _MBLK


def _col_spec():
    return pl.BlockSpec((_MBLK, 1), lambda i: (i, 0))


def _mat_spec(d):
    return pl.BlockSpec((_MBLK, d), lambda i: (i, 0))


def _full_spec(r, c):
    return pl.BlockSpec((r, c), lambda i: (0, 0))


def _tc1_body(p0, p1, m, x, w1, g1, dv):
    deg = p0[...] + p1[...] + m[...]
    dinv = jnp.where(deg > 0, lax.rsqrt(deg), 0.0)
    g1[...] = (dinv * jnp.dot(x[...], w1[...],
                              preferred_element_type=jnp.float32)
               ).astype(jnp.bfloat16)
    dv[...] = dinv


def _tc1(p0, p1, m, x, w1):
    return pl.pallas_call(
        _tc1_body,
        grid=(_GRID,),
        in_specs=[_col_spec(), _col_spec(), _col_spec(), _mat_spec(128),
                  _full_spec(128, 128)],
        out_specs=[_mat_spec(128), _col_spec()],
        out_shape=[jax.ShapeDtypeStruct((NPAD, 128), jnp.bfloat16),
                   jax.ShapeDtypeStruct((NPAD, 1), jnp.float32)],
    )(p0, p1, m, x, w1)


def _tc2_body(q, g1, dv, b1, w2, g2):
    dinv = dv[...]
    s = q[...].astype(jnp.float32) + g1[...].astype(jnp.float32)
    h1 = jnp.maximum(dinv * s + b1[...], 0.0)
    g2[...] = (dinv * jnp.dot(h1, w2[...],
                              preferred_element_type=jnp.float32)
               ).astype(jnp.bfloat16)


def _tc2(q, g1, dv, b1, w2):
    return pl.pallas_call(
        _tc2_body,
        grid=(_GRID,),
        in_specs=[_mat_spec(128), _mat_spec(128), _col_spec(),
                  _full_spec(1, 128), _full_spec(128, 64)],
        out_specs=_mat_spec(64),
        out_shape=jax.ShapeDtypeStruct((NPAD, 64), jnp.bfloat16),
    )(q, g1, dv, b1, w2)


def _tc3_body(r, g2, dv, b2, out):
    s = r[...].astype(jnp.float32) + g2[...].astype(jnp.float32)
    out[...] = jnp.maximum(dv[...] * s + b2[...], 0.0)


def _tc3(r, g2, dv, b2):
    return pl.pallas_call(
        _tc3_body,
        grid=(_GRID,),
        in_specs=[_mat_spec(64), _mat_spec(64), _col_spec(),
                  _full_spec(1, 64)],
        out_specs=_mat_spec(64),
        out_shape=jax.ShapeDtypeStruct((NPAD, 64), jnp.float32),
    )(r, g2, dv, b2)


@jax.jit
def kernel(x, edge_index, W1, b1, W2, b2):
    src = edge_index[0].astype(jnp.int32)
    dst = edge_index[1].astype(jnp.int32)
    pad = EPAD - E
    src2d = jnp.concatenate(
        [src, jnp.full((pad,), PAD_SRC, jnp.int32)]).reshape(ROWS2D, CHUNK)
    dst2d = jnp.concatenate(
        [dst, jnp.full((pad,), PAD_DST, jnp.int32)]).reshape(ROWS2D, CHUNK)
    xp = jnp.pad(x, ((0, NPAD - N), (0, 0)))
    maskc = (jnp.arange(NPAD) < N).astype(jnp.float32).reshape(NPAD, 1)

    degp = _degree_kernel(dst2d)
    p0 = degp[0].reshape(NPAD, 1)
    p1 = degp[1].reshape(NPAD, 1)
    g1, dv = _tc1(p0, p1, maskc, xp, W1)

    s1 = _propagate(g1, src2d, dst2d, 64)
    g2 = _tc2(s1, g1, dv, b1.reshape(1, 128), W2)

    s2 = _propagate(g2, src2d, dst2d, 32)
    out = _tc3(s2, g2, dv, b2.reshape(1, 64))
    return out[:N]
